# baseline (speedup 1.0000x reference)
import jax
import jax.numpy as jnp
from jax.experimental import pallas as pl

HS = 2048
NB = 32
RANK = 8
GRID_N = 8
SIGMA = 1.0
ROW_TILE = 512
QPAD = 128
ZW = RANK * NB
CW = ZW + QPAD


def _fused_kernel(x_ref, prior_ref, comb_ref, colsum_ref, bias_ref,
                  centers_ref, up_ref, scal_ref, out_ref):
    x = x_ref[...]
    rps = scal_ref[0, 0]
    res = scal_ref[0, 1]

    zq = jnp.dot(x, comb_ref[...], preferred_element_type=jnp.float32)
    z = zq[:, :ZW]
    xq = zq[:, ZW:]

    mu = xq[:, 3:4] * (1.0 / HS)
    var = jnp.mean(x * x, axis=1, keepdims=True) - mu * mu
    rs = jax.lax.rsqrt(var + 1e-5)

    qraw = rs * (xq - mu * colsum_ref[...]) + bias_ref[...]
    q = jax.nn.sigmoid(qraw) * float(GRID_N - 1)
    col = jax.lax.broadcasted_iota(jnp.int32, q.shape, 1)
    q = jnp.where(col < 3, q, 0.0)
    c = centers_ref[...]
    qn = jnp.sum(q * q, axis=1, keepdims=True)
    cn = jnp.sum(c * c, axis=1)[None, :]
    qc = jnp.dot(q, c.T, preferred_element_type=jnp.float32)
    d2 = qn + cn - 2.0 * qc
    spatial = jnp.exp(-d2 / (2.0 * SIGMA * SIGMA))

    prior = jnp.maximum(prior_ref[...], 0.0)
    prior = prior / jnp.maximum(jnp.sum(prior, axis=1, keepdims=True), 1e-6)
    prior_bias = jnp.clip(jnp.log(prior + 1e-6), -6.0, 0.0)
    fused = spatial + rps * prior_bias

    iota = jax.lax.broadcasted_iota(jnp.int32, fused.shape, 1)
    m1 = jnp.max(fused, axis=1, keepdims=True)
    i1 = jnp.min(jnp.where(fused == m1, iota, NB), axis=1, keepdims=True)
    oh1 = iota == i1
    masked = jnp.where(oh1, -jnp.inf, fused)
    m2 = jnp.max(masked, axis=1, keepdims=True)
    i2 = jnp.min(jnp.where(masked == m2, iota, NB), axis=1, keepdims=True)
    oh2 = iota == i2
    e2 = jnp.exp(m2 - m1)
    w1 = 1.0 / (1.0 + e2)
    w2 = e2 * w1
    wfull = jnp.where(oh1, w1, 0.0) + jnp.where(oh2, w2, 0.0)

    wtile = jnp.concatenate([wfull] * RANK, axis=1)
    delta = jnp.dot(z * wtile, up_ref[...], preferred_element_type=jnp.float32)
    out_ref[...] = x + res * delta


def kernel(hidden_states, route_prior, W_proj, b_proj, block_centers, down_w,
           up_w, route_prior_scale, residual_scale):
    b, s, h = hidden_states.shape
    rows = b * s
    flat = hidden_states.reshape(rows, h)

    down_all = down_w.transpose(1, 2, 0).reshape(h, ZW)
    wproj_pad = jnp.zeros((h, QPAD), jnp.float32).at[:, :3].set(W_proj.T)
    wproj_pad = wproj_pad.at[:, 3].set(1.0)
    comb = jnp.concatenate([down_all, wproj_pad], axis=1)
    colsum = jnp.zeros((1, QPAD), jnp.float32).at[0, :3].set(jnp.sum(W_proj.T[:, :3], axis=0))
    bias_pad = jnp.zeros((1, QPAD), jnp.float32).at[0, :3].set(b_proj)
    centers_pad = jnp.zeros((NB, QPAD), jnp.float32).at[:, :3].set(block_centers)
    up_all = up_w.transpose(1, 0, 2).reshape(ZW, h)
    scal = jnp.stack([route_prior_scale, residual_scale]).reshape(1, 2).astype(jnp.float32)

    grid = rows // ROW_TILE

    out = pl.pallas_call(
        _fused_kernel,
        grid=(grid,),
        in_specs=[
            pl.BlockSpec((ROW_TILE, h), lambda i: (i, 0)),
            pl.BlockSpec((ROW_TILE, NB), lambda i: (i, 0)),
            pl.BlockSpec((h, CW), lambda i: (0, 0)),
            pl.BlockSpec((1, QPAD), lambda i: (0, 0)),
            pl.BlockSpec((1, QPAD), lambda i: (0, 0)),
            pl.BlockSpec((NB, QPAD), lambda i: (0, 0)),
            pl.BlockSpec((ZW, h), lambda i: (0, 0)),
            pl.BlockSpec((1, 2), lambda i: (0, 0)),
        ],
        out_specs=pl.BlockSpec((ROW_TILE, h), lambda i: (i, 0)),
        out_shape=jax.ShapeDtypeStruct((rows, h), jnp.float32),
    )(flat, route_prior, comb, colsum, bias_pad, centers_pad, up_all, scal)

    return out.reshape(b, s, h)


# speedup vs baseline: 1.0876x; 1.0876x over previous
import jax
import jax.numpy as jnp
from jax.experimental import pallas as pl

HS = 2048
NB = 32
RANK = 8
GRID_N = 8
SIGMA = 1.0
ROW_TILE = 512
QPAD = 128
ZW = RANK * NB


def _fused_kernel(x_ref, prior_ref, wproj_ref, colsum_ref, bias_ref,
                  centers_ref, down_ref, up_ref, scal_ref, out_ref):
    x = x_ref[...]
    rps = scal_ref[0, 0]
    res = scal_ref[0, 1]

    xq = jnp.dot(x, wproj_ref[...], preferred_element_type=jnp.float32)
    mu = xq[:, 3:4] * (1.0 / HS)
    var = jnp.mean(x * x, axis=1, keepdims=True) - mu * mu
    rs = jax.lax.rsqrt(var + 1e-5)

    qraw = rs * (xq - mu * colsum_ref[...]) + bias_ref[...]
    q = jax.nn.sigmoid(qraw) * float(GRID_N - 1)
    col = jax.lax.broadcasted_iota(jnp.int32, q.shape, 1)
    q = jnp.where(col < 3, q, 0.0)
    c = centers_ref[...]
    qn = jnp.sum(q * q, axis=1, keepdims=True)
    cn = jnp.sum(c * c, axis=1)[None, :]
    qc = jnp.dot(q, c.T, preferred_element_type=jnp.float32)
    d2 = qn + cn - 2.0 * qc
    spatial = jnp.exp(-d2 / (2.0 * SIGMA * SIGMA))

    prior = jnp.maximum(prior_ref[...], 0.0)
    prior = prior / jnp.maximum(jnp.sum(prior, axis=1, keepdims=True), 1e-6)
    prior_bias = jnp.clip(jnp.log(prior + 1e-6), -6.0, 0.0)
    fused = spatial + rps * prior_bias

    iota = jax.lax.broadcasted_iota(jnp.int32, fused.shape, 1)
    m1 = jnp.max(fused, axis=1, keepdims=True)
    i1 = jnp.min(jnp.where(fused == m1, iota, NB), axis=1, keepdims=True)
    oh1 = iota == i1
    masked = jnp.where(oh1, -jnp.inf, fused)
    m2 = jnp.max(masked, axis=1, keepdims=True)
    i2 = jnp.min(jnp.where(masked == m2, iota, NB), axis=1, keepdims=True)
    oh2 = iota == i2
    e2 = jnp.exp(m2 - m1)
    w1 = 1.0 / (1.0 + e2)
    w2 = e2 * w1
    wfull = jnp.where(oh1, w1, 0.0) + jnp.where(oh2, w2, 0.0)

    z = jnp.dot(x.astype(jnp.bfloat16), down_ref[...],
                preferred_element_type=jnp.float32)
    wtile = jnp.concatenate([wfull] * RANK, axis=1)
    zw = (z * wtile).astype(jnp.bfloat16)
    delta = jnp.dot(zw, up_ref[...], preferred_element_type=jnp.float32)
    out_ref[...] = x + res * delta


def kernel(hidden_states, route_prior, W_proj, b_proj, block_centers, down_w,
           up_w, route_prior_scale, residual_scale):
    b, s, h = hidden_states.shape
    rows = b * s
    flat = hidden_states.reshape(rows, h)

    wproj_pad = jnp.zeros((h, QPAD), jnp.float32).at[:, :3].set(W_proj.T)
    wproj_pad = wproj_pad.at[:, 3].set(1.0)
    colsum = jnp.zeros((1, QPAD), jnp.float32).at[0, :3].set(jnp.sum(W_proj, axis=1))
    bias_pad = jnp.zeros((1, QPAD), jnp.float32).at[0, :3].set(b_proj)
    centers_pad = jnp.zeros((NB, QPAD), jnp.float32).at[:, :3].set(block_centers)
    down_all = down_w.transpose(1, 2, 0).reshape(h, ZW).astype(jnp.bfloat16)
    up_all = up_w.transpose(1, 0, 2).reshape(ZW, h).astype(jnp.bfloat16)
    scal = jnp.stack([route_prior_scale, residual_scale]).reshape(1, 2).astype(jnp.float32)

    grid = rows // ROW_TILE

    out = pl.pallas_call(
        _fused_kernel,
        grid=(grid,),
        in_specs=[
            pl.BlockSpec((ROW_TILE, h), lambda i: (i, 0)),
            pl.BlockSpec((ROW_TILE, NB), lambda i: (i, 0)),
            pl.BlockSpec((h, QPAD), lambda i: (0, 0)),
            pl.BlockSpec((1, QPAD), lambda i: (0, 0)),
            pl.BlockSpec((1, QPAD), lambda i: (0, 0)),
            pl.BlockSpec((NB, QPAD), lambda i: (0, 0)),
            pl.BlockSpec((h, ZW), lambda i: (0, 0)),
            pl.BlockSpec((ZW, h), lambda i: (0, 0)),
            pl.BlockSpec((1, 2), lambda i: (0, 0)),
        ],
        out_specs=pl.BlockSpec((ROW_TILE, h), lambda i: (i, 0)),
        out_shape=jax.ShapeDtypeStruct((rows, h), jnp.float32),
    )(flat, route_prior, wproj_pad, colsum, bias_pad, centers_pad, down_all,
      up_all, scal)

    return out.reshape(b, s, h)


# speedup vs baseline: 1.1811x; 1.0860x over previous
import jax
import jax.numpy as jnp
from jax.experimental import pallas as pl

HS = 2048
NB = 32
RANK = 8
GRID_N = 8
SIGMA = 1.0
ROW_TILE = 1024
QPAD = 128


def _fused_kernel(x_ref, prior_ref, wproj_ref, bias_ref, centers_ref,
                  down_ref, up_ref, scal_ref, out_ref):
    x = x_ref[...]
    rps = scal_ref[0, 0]
    res = scal_ref[0, 1]

    mu = jnp.mean(x, axis=1, keepdims=True)
    var = jnp.mean(x * x, axis=1, keepdims=True) - mu * mu
    ln = (x - mu) * jax.lax.rsqrt(var + 1e-5)

    qraw = jnp.dot(ln, wproj_ref[...], preferred_element_type=jnp.float32)
    qraw = qraw + bias_ref[...]
    q = jax.nn.sigmoid(qraw) * float(GRID_N - 1)
    col = jax.lax.broadcasted_iota(jnp.int32, q.shape, 1)
    q = jnp.where(col < 3, q, 0.0)
    c = centers_ref[...]
    qn = jnp.sum(q * q, axis=1, keepdims=True)
    cn = jnp.sum(c * c, axis=1)[None, :]
    qc = jnp.dot(q, c.T, preferred_element_type=jnp.float32)
    d2 = qn + cn - 2.0 * qc
    spatial = jnp.exp(-d2 / (2.0 * SIGMA * SIGMA))

    prior = jnp.maximum(prior_ref[...], 0.0)
    prior = prior / jnp.maximum(jnp.sum(prior, axis=1, keepdims=True), 1e-6)
    prior_bias = jnp.clip(jnp.log(prior + 1e-6), -6.0, 0.0)
    fused = spatial + rps * prior_bias

    iota = jax.lax.broadcasted_iota(jnp.int32, fused.shape, 1)
    m1 = jnp.max(fused, axis=1, keepdims=True)
    i1 = jnp.min(jnp.where(fused == m1, iota, NB), axis=1, keepdims=True)
    oh1 = iota == i1
    masked = jnp.where(oh1, -jnp.inf, fused)
    m2 = jnp.max(masked, axis=1, keepdims=True)
    i2 = jnp.min(jnp.where(masked == m2, iota, NB), axis=1, keepdims=True)
    oh2 = iota == i2
    e2 = jnp.exp(m2 - m1)
    w1 = 1.0 / (1.0 + e2)
    w2 = e2 * w1
    wfull = jnp.where(oh1, w1, 0.0) + jnp.where(oh2, w2, 0.0)

    z = jnp.dot(x, down_ref[...], preferred_element_type=jnp.float32)
    wtile = jnp.concatenate([wfull] * RANK, axis=1)
    delta = jnp.dot(z * wtile, up_ref[...], preferred_element_type=jnp.float32)
    out_ref[...] = x + res * delta


def kernel(hidden_states, route_prior, W_proj, b_proj, block_centers, down_w,
           up_w, route_prior_scale, residual_scale):
    b, s, h = hidden_states.shape
    rows = b * s
    flat = hidden_states.reshape(rows, h)

    wproj_pad = jnp.zeros((h, QPAD), jnp.float32).at[:, :3].set(W_proj.T)
    bias_pad = jnp.zeros((1, QPAD), jnp.float32).at[0, :3].set(b_proj)
    centers_pad = jnp.zeros((NB, QPAD), jnp.float32).at[:, :3].set(block_centers)
    down_all = down_w.transpose(1, 2, 0).reshape(h, RANK * NB)
    up_all = up_w.transpose(1, 0, 2).reshape(RANK * NB, h)
    scal = jnp.stack([route_prior_scale, residual_scale]).reshape(1, 2).astype(jnp.float32)

    grid = rows // ROW_TILE

    out = pl.pallas_call(
        _fused_kernel,
        grid=(grid,),
        in_specs=[
            pl.BlockSpec((ROW_TILE, h), lambda i: (i, 0)),
            pl.BlockSpec((ROW_TILE, NB), lambda i: (i, 0)),
            pl.BlockSpec((h, QPAD), lambda i: (0, 0)),
            pl.BlockSpec((1, QPAD), lambda i: (0, 0)),
            pl.BlockSpec((NB, QPAD), lambda i: (0, 0)),
            pl.BlockSpec((h, RANK * NB), lambda i: (0, 0)),
            pl.BlockSpec((RANK * NB, h), lambda i: (0, 0)),
            pl.BlockSpec((1, 2), lambda i: (0, 0)),
        ],
        out_specs=pl.BlockSpec((ROW_TILE, h), lambda i: (i, 0)),
        out_shape=jax.ShapeDtypeStruct((rows, h), jnp.float32),
    )(flat, route_prior, wproj_pad, bias_pad, centers_pad, down_all, up_all, scal)

    return out.reshape(b, s, h)


# speedup vs baseline: 1.2223x; 1.0348x over previous
import jax
import jax.numpy as jnp
from jax.experimental import pallas as pl

HS = 2048
NB = 32
RANK = 8
GRID_N = 8
SIGMA = 1.0
ROW_TILE = 1024
QPAD = 128


def _fused_kernel(x_ref, prior_ref, wproj_ref, bias_ref, centers_ref,
                  down_ref, up_ref, scal_ref, out_ref):
    x = x_ref[...]
    rps = scal_ref[0, 0]
    res = scal_ref[0, 1]

    mu = jnp.mean(x, axis=1, keepdims=True)
    var = jnp.mean(x * x, axis=1, keepdims=True) - mu * mu
    ln = (x - mu) * jax.lax.rsqrt(var + 1e-5)

    qraw = jnp.dot(ln, wproj_ref[...], preferred_element_type=jnp.float32)
    qraw = qraw + bias_ref[...]
    q = jax.nn.sigmoid(qraw) * float(GRID_N - 1)
    col = jax.lax.broadcasted_iota(jnp.int32, q.shape, 1)
    q = jnp.where(col < 3, q, 0.0)
    c = centers_ref[...]
    qn = jnp.sum(q * q, axis=1, keepdims=True)
    cn = jnp.sum(c * c, axis=1)[None, :]
    qc = jnp.dot(q, c.T, preferred_element_type=jnp.float32)
    d2 = qn + cn - 2.0 * qc
    spatial = jnp.exp(-d2 / (2.0 * SIGMA * SIGMA))

    prior = jnp.maximum(prior_ref[...], 0.0)
    prior = prior / jnp.maximum(jnp.sum(prior, axis=1, keepdims=True), 1e-6)
    prior_bias = jnp.clip(jnp.log(prior + 1e-6), -6.0, 0.0)
    fused = spatial + rps * prior_bias

    iota = jax.lax.broadcasted_iota(jnp.int32, fused.shape, 1)
    m1 = jnp.max(fused, axis=1, keepdims=True)
    i1 = jnp.min(jnp.where(fused == m1, iota, NB), axis=1, keepdims=True)
    oh1 = iota == i1
    masked = jnp.where(oh1, -jnp.inf, fused)
    m2 = jnp.max(masked, axis=1, keepdims=True)
    i2 = jnp.min(jnp.where(masked == m2, iota, NB), axis=1, keepdims=True)
    oh2 = iota == i2
    e2 = jnp.exp(m2 - m1)
    w1 = 1.0 / (1.0 + e2)
    w2 = e2 * w1
    wfull = jnp.where(oh1, w1, 0.0) + jnp.where(oh2, w2, 0.0)

    z = jnp.dot(x, down_ref[...], preferred_element_type=jnp.float32)
    erow = jax.lax.broadcasted_iota(jnp.int32, (NB, RANK * NB), 0)
    ecol = jax.lax.broadcasted_iota(jnp.int32, (NB, RANK * NB), 1)
    expand = (erow == ecol // RANK).astype(jnp.float32)
    wexp = jnp.dot(wfull, expand, preferred_element_type=jnp.float32)
    delta = jnp.dot(z * wexp, up_ref[...], preferred_element_type=jnp.float32)
    out_ref[...] = x + res * delta


def kernel(hidden_states, route_prior, W_proj, b_proj, block_centers, down_w,
           up_w, route_prior_scale, residual_scale):
    b, s, h = hidden_states.shape
    rows = b * s
    flat = hidden_states.reshape(rows, h)

    wproj_pad = jnp.zeros((h, QPAD), jnp.float32).at[:, :3].set(W_proj.T)
    bias_pad = jnp.zeros((1, QPAD), jnp.float32).at[0, :3].set(b_proj)
    centers_pad = jnp.zeros((NB, QPAD), jnp.float32).at[:, :3].set(block_centers)
    down_all = down_w.transpose(1, 0, 2).reshape(h, RANK * NB)
    up_all = up_w.reshape(RANK * NB, h)
    scal = jnp.stack([route_prior_scale, residual_scale]).reshape(1, 2).astype(jnp.float32)

    grid = rows // ROW_TILE

    out = pl.pallas_call(
        _fused_kernel,
        grid=(grid,),
        in_specs=[
            pl.BlockSpec((ROW_TILE, h), lambda i: (i, 0)),
            pl.BlockSpec((ROW_TILE, NB), lambda i: (i, 0)),
            pl.BlockSpec((h, QPAD), lambda i: (0, 0)),
            pl.BlockSpec((1, QPAD), lambda i: (0, 0)),
            pl.BlockSpec((NB, QPAD), lambda i: (0, 0)),
            pl.BlockSpec((h, RANK * NB), lambda i: (0, 0)),
            pl.BlockSpec((RANK * NB, h), lambda i: (0, 0)),
            pl.BlockSpec((1, 2), lambda i: (0, 0)),
        ],
        out_specs=pl.BlockSpec((ROW_TILE, h), lambda i: (i, 0)),
        out_shape=jax.ShapeDtypeStruct((rows, h), jnp.float32),
    )(flat, route_prior, wproj_pad, bias_pad, centers_pad, down_all, up_all, scal)

    return out.reshape(b, s, h)


# speedup vs baseline: 1.2930x; 1.0578x over previous
import jax
import jax.numpy as jnp
from jax.experimental import pallas as pl

HS = 2048
NB = 32
RANK = 8
GRID_N = 8
SIGMA = 1.0
ROW_TILE = 1024
QPAD = 8

_TRANS_B = (((1,), (1,)), ((), ()))


def _fused_kernel(x_ref, prior_ref, wproj_ref, bias_ref, caug_ref,
                  down_ref, up_ref, scal_ref, out_ref):
    x = x_ref[...]
    rps = scal_ref[0, 0]
    res = scal_ref[0, 1]

    mu = jnp.mean(x, axis=1, keepdims=True)
    var = jnp.mean(x * x, axis=1, keepdims=True) - mu * mu
    ln = (x - mu) * jax.lax.rsqrt(var + 1e-5)

    qraw = jax.lax.dot_general(ln, wproj_ref[...], _TRANS_B,
                               preferred_element_type=jnp.float32)
    qraw = qraw + bias_ref[...]
    col = jax.lax.broadcasted_iota(jnp.int32, qraw.shape, 1)
    q = jnp.where(col < 3, jax.nn.sigmoid(qraw) * float(GRID_N - 1), 0.0)
    qn = jnp.sum(q * q, axis=1, keepdims=True)
    q_aug = q + (col == 3).astype(jnp.float32)
    d2 = qn + jax.lax.dot_general(q_aug, caug_ref[...], _TRANS_B,
                                  preferred_element_type=jnp.float32)
    spatial = jnp.exp(d2 * (-1.0 / (2.0 * SIGMA * SIGMA)))

    prior = jnp.maximum(prior_ref[...], 0.0)
    prior = prior / jnp.maximum(jnp.sum(prior, axis=1, keepdims=True), 1e-6)
    prior_bias = jnp.clip(jnp.log(prior + 1e-6), -6.0, 0.0)
    fused = spatial + rps * prior_bias

    iota = jax.lax.broadcasted_iota(jnp.int32, fused.shape, 1)
    m1 = jnp.max(fused, axis=1, keepdims=True)
    i1 = jnp.min(jnp.where(fused == m1, iota, NB), axis=1, keepdims=True)
    oh1 = iota == i1
    masked = jnp.where(oh1, -jnp.inf, fused)
    m2 = jnp.max(masked, axis=1, keepdims=True)
    i2 = jnp.min(jnp.where(masked == m2, iota, NB), axis=1, keepdims=True)
    oh2 = iota == i2
    e2 = jnp.exp(m2 - m1)
    w1 = 1.0 / (1.0 + e2)
    w2 = e2 * w1
    wfull = jnp.where(oh1, w1, 0.0) + jnp.where(oh2, w2, 0.0)

    z = jnp.dot(x, down_ref[...], preferred_element_type=jnp.float32)
    erow = jax.lax.broadcasted_iota(jnp.int32, (NB, RANK * NB), 0)
    ecol = jax.lax.broadcasted_iota(jnp.int32, (NB, RANK * NB), 1)
    expand = (erow == ecol // RANK).astype(jnp.float32)
    wexp = jnp.dot(wfull, expand, preferred_element_type=jnp.float32)
    delta = jnp.dot(z * wexp, up_ref[...], preferred_element_type=jnp.float32)
    out_ref[...] = x + res * delta


def kernel(hidden_states, route_prior, W_proj, b_proj, block_centers, down_w,
           up_w, route_prior_scale, residual_scale):
    b, s, h = hidden_states.shape
    rows = b * s
    flat = hidden_states.reshape(rows, h)

    wp = jnp.pad(W_proj, ((0, QPAD - 3), (0, 0)))
    bias_pad = jnp.pad(b_proj, (0, QPAD - 3)).reshape(1, QPAD)
    caug = jnp.concatenate(
        [-2.0 * block_centers,
         jnp.sum(block_centers * block_centers, axis=1, keepdims=True),
         jnp.zeros((NB, QPAD - 4), jnp.float32)], axis=1)
    down_all = down_w.transpose(1, 0, 2).reshape(h, RANK * NB)
    up_all = up_w.reshape(RANK * NB, h)
    scal = jnp.stack([route_prior_scale, residual_scale]).reshape(1, 2).astype(jnp.float32)

    grid = rows // ROW_TILE

    out = pl.pallas_call(
        _fused_kernel,
        grid=(grid,),
        in_specs=[
            pl.BlockSpec((ROW_TILE, h), lambda i: (i, 0)),
            pl.BlockSpec((ROW_TILE, NB), lambda i: (i, 0)),
            pl.BlockSpec((QPAD, h), lambda i: (0, 0)),
            pl.BlockSpec((1, QPAD), lambda i: (0, 0)),
            pl.BlockSpec((NB, QPAD), lambda i: (0, 0)),
            pl.BlockSpec((h, RANK * NB), lambda i: (0, 0)),
            pl.BlockSpec((RANK * NB, h), lambda i: (0, 0)),
            pl.BlockSpec((1, 2), lambda i: (0, 0)),
        ],
        out_specs=pl.BlockSpec((ROW_TILE, h), lambda i: (i, 0)),
        out_shape=jax.ShapeDtypeStruct((rows, h), jnp.float32),
    )(flat, route_prior, wp, bias_pad, caug, down_all, up_all, scal)

    return out.reshape(b, s, h)


# speedup vs baseline: 1.3048x; 1.0092x over previous
import jax
import jax.numpy as jnp
from jax.experimental import pallas as pl

HS = 2048
NB = 32
RANK = 8
GRID_N = 8
SIGMA = 1.0
ROW_TILE = 1024
QPAD = 8

_TRANS_B = (((1,), (1,)), ((), ()))


def _fused_kernel(x_ref, prior_ref, wproj_ref, sb_ref, caug_ref,
                  down_ref, up_ref, scal_ref, out_ref):
    x = x_ref[...]
    rps = scal_ref[0, 0]
    res = scal_ref[0, 1]

    xq = jax.lax.dot_general(x, wproj_ref[...], _TRANS_B,
                             preferred_element_type=jnp.float32)
    mu = xq[:, 3:4] * (1.0 / HS)
    var = jnp.mean(x * x, axis=1, keepdims=True) - mu * mu
    rs = jax.lax.rsqrt(var + 1e-5)
    qraw = rs * (xq - mu * sb_ref[1:2, :]) + sb_ref[0:1, :]
    col = jax.lax.broadcasted_iota(jnp.int32, qraw.shape, 1)
    q = jnp.where(col < 3, jax.nn.sigmoid(qraw) * float(GRID_N - 1), 0.0)
    qn = jnp.sum(q * q, axis=1, keepdims=True)
    q_aug = q + (col == 3).astype(jnp.float32)
    d2 = qn + jax.lax.dot_general(q_aug, caug_ref[...], _TRANS_B,
                                  preferred_element_type=jnp.float32)
    spatial = jnp.exp(d2 * (-1.0 / (2.0 * SIGMA * SIGMA)))

    prior = jnp.maximum(prior_ref[...], 0.0)
    prior = prior / jnp.maximum(jnp.sum(prior, axis=1, keepdims=True), 1e-6)
    prior_bias = jnp.clip(jnp.log(prior + 1e-6), -6.0, 0.0)
    fused = spatial + rps * prior_bias

    iota = jax.lax.broadcasted_iota(jnp.int32, fused.shape, 1)
    m1 = jnp.max(fused, axis=1, keepdims=True)
    i1 = jnp.min(jnp.where(fused == m1, iota, NB), axis=1, keepdims=True)
    oh1 = iota == i1
    masked = jnp.where(oh1, -jnp.inf, fused)
    m2 = jnp.max(masked, axis=1, keepdims=True)
    i2 = jnp.min(jnp.where(masked == m2, iota, NB), axis=1, keepdims=True)
    oh2 = iota == i2
    e2 = jnp.exp(m2 - m1)
    w1 = 1.0 / (1.0 + e2)
    w2 = e2 * w1
    wfull = jnp.where(oh1, w1, 0.0) + jnp.where(oh2, w2, 0.0)

    z = jnp.dot(x, down_ref[...], preferred_element_type=jnp.float32)
    erow = jax.lax.broadcasted_iota(jnp.int32, (NB, RANK * NB), 0)
    ecol = jax.lax.broadcasted_iota(jnp.int32, (NB, RANK * NB), 1)
    expand = (erow == ecol // RANK).astype(jnp.float32)
    wexp = jnp.dot(wfull, expand, preferred_element_type=jnp.float32)
    delta = jnp.dot(z * wexp, up_ref[...], preferred_element_type=jnp.float32)
    out_ref[...] = x + res * delta


def kernel(hidden_states, route_prior, W_proj, b_proj, block_centers, down_w,
           up_w, route_prior_scale, residual_scale):
    b, s, h = hidden_states.shape
    rows = b * s
    flat = hidden_states.reshape(rows, h)

    wp = jnp.pad(W_proj, ((0, QPAD - 3), (0, 0)))
    wp = wp.at[3, :].set(1.0)
    bias_pad = jnp.pad(b_proj, (0, QPAD - 3)).reshape(1, QPAD)
    colsum = jnp.pad(jnp.sum(W_proj, axis=1), (0, QPAD - 3)).reshape(1, QPAD)
    sb = jnp.concatenate([bias_pad, colsum], axis=0)
    caug = jnp.concatenate(
        [-2.0 * block_centers,
         jnp.sum(block_centers * block_centers, axis=1, keepdims=True),
         jnp.zeros((NB, QPAD - 4), jnp.float32)], axis=1)
    down_all = down_w.transpose(1, 0, 2).reshape(h, RANK * NB)
    up_all = up_w.reshape(RANK * NB, h)
    scal = jnp.stack([route_prior_scale, residual_scale]).reshape(1, 2).astype(jnp.float32)

    grid = rows // ROW_TILE

    out = pl.pallas_call(
        _fused_kernel,
        grid=(grid,),
        in_specs=[
            pl.BlockSpec((ROW_TILE, h), lambda i: (i, 0)),
            pl.BlockSpec((ROW_TILE, NB), lambda i: (i, 0)),
            pl.BlockSpec((QPAD, h), lambda i: (0, 0)),
            pl.BlockSpec((2, QPAD), lambda i: (0, 0)),
            pl.BlockSpec((NB, QPAD), lambda i: (0, 0)),
            pl.BlockSpec((h, RANK * NB), lambda i: (0, 0)),
            pl.BlockSpec((RANK * NB, h), lambda i: (0, 0)),
            pl.BlockSpec((1, 2), lambda i: (0, 0)),
        ],
        out_specs=pl.BlockSpec((ROW_TILE, h), lambda i: (i, 0)),
        out_shape=jax.ShapeDtypeStruct((rows, h), jnp.float32),
    )(flat, route_prior, wp, sb, caug, down_all, up_all, scal)

    return out.reshape(b, s, h)
